# Initial kernel scaffold; baseline (speedup 1.0000x reference)
import jax
import jax.numpy as jnp
from jax.experimental import pallas as pl


def kernel(x, edge_index, batch, W1, b1, W2, b2, Wg1, bg1, as1, ad1, Wg2, bg2, as2, ad2, W3a, b3a, W3b, b3b, Wout, bout):
    raise NotImplementedError("write your pallas kernel here")


# speedup vs baseline: 4.6381x; 4.6381x over previous
import functools

import jax
import jax.numpy as jnp
from jax import lax
from jax.experimental import pallas as pl
from jax.experimental.pallas import tpu as pltpu
import jax.experimental.pallas.tpu_sc as plsc

f32 = jnp.float32
i32 = jnp.int32

N = 10000
NP = 10240
B = 64
E = 320000
E2 = E + N
CH = 128
SUP = 8
EPW32 = 11264
EPW16 = 22528
E2P = 360448
NSUB = 16
NCORE = 2
NB = 2
RPS = NP // NSUB
R = 1024
GRID = NP // R


def _mesh():
  return plsc.VectorSubcoreMesh(
      core_axis_name="c", subcore_axis_name="s",
      num_cores=NCORE, num_subcores=NSUB)


def _zero_rows(buf, nrows, ncolv):
  def body(i, carry):
    for k in range(ncolv):
      buf[i, pl.ds(k * 16, 16)] = jnp.zeros((16,), f32)
    return carry
  lax.fori_loop(0, nrows, body, 0)


def _zero_shared(acc, zsrc, s):
  for k in range(RPS // CH):
    pltpu.sync_copy(zsrc, acc.at[pl.ds(s * RPS + k * CH, CH)])


def _make_deg():
  @functools.partial(
      pl.kernel,
      out_type=(jax.ShapeDtypeStruct((NP,), f32),
                jax.ShapeDtypeStruct((NP,), f32)),
      mesh=_mesh(),
      scratch_types=[
          pltpu.VMEM((EPW32 // CH, CH), i32),
          pltpu.VMEM((CH,), f32),
          pltpu.VMEM((RPS,), f32),
          pltpu.VMEM_SHARED((NP,), f32),
      ] + [pltpu.SemaphoreType.DMA] * NB,
  )
  def deg(dst2, dega, degb, idx_v, ones_v, zb, acc, *sems):
    c = lax.axis_index("c")
    s = lax.axis_index("s")
    wid = c * NSUB + s
    nch = EPW32 // CH
    for k in range(CH // 16):
      ones_v[pl.ds(k * 16, 16)] = jnp.ones((16,), f32)
    for k in range(RPS // 16):
      zb[pl.ds(k * 16, 16)] = jnp.zeros((16,), f32)
    pltpu.sync_copy(zb, acc.at[pl.ds(s * RPS, RPS)])
    plsc.subcore_barrier()

    pltpu.sync_copy(dst2.at[pl.ds(wid * nch, nch)], idx_v)
    def rnd(r, carry):
      for b in range(NB):
        j = r * NB + b
        @pl.when(r > 0)
        def _():
          pltpu.make_async_copy(ones_v, acc.at[pl.ds(0, CH)],
                                sems[b]).wait()
        pltpu.async_copy(ones_v, acc.at[idx_v.at[j]], sems[b], add=True)
      return carry
    lax.fori_loop(0, nch // NB, rnd, 0)
    for b in range(NB):
      pltpu.make_async_copy(ones_v, acc.at[pl.ds(0, CH)], sems[b]).wait()
    plsc.subcore_barrier()

    @pl.when(c == 0)
    def _():
      pltpu.sync_copy(acc.at[pl.ds(s * RPS, RPS)],
                      dega.at[pl.ds(s * RPS, RPS)])
    @pl.when(c == 1)
    def _():
      pltpu.sync_copy(acc.at[pl.ds(s * RPS, RPS)],
                      degb.at[pl.ds(s * RPS, RPS)])
  return deg


def _make_seg32():
  @functools.partial(
      pl.kernel,
      out_type=jax.ShapeDtypeStruct((NCORE, NP, 128), f32),
      mesh=_mesh(),
      scratch_types=[
          pltpu.VMEM((2, SUP, CH), i32),
          pltpu.VMEM((2, SUP, CH), i32),
      ] + [pltpu.VMEM((CH, 128), f32)] * NB + [
          pltpu.VMEM_SHARED((NP, 128), f32),
      ] + [pltpu.SemaphoreType.DMA] * (2 * NB + 1),
  )
  def seg32(table, src2, dst2, out, idxs_v, idxd_v, *rest):
    rows = rest[:NB]
    acc = rest[NB]
    gs = rest[NB + 1:NB + 1 + NB]
    ss = rest[NB + 1 + NB:NB + 1 + 2 * NB]
    isem = rest[NB + 1 + 2 * NB]
    c = lax.axis_index("c")
    s = lax.axis_index("s")
    wid = c * NSUB + s
    nch = EPW32 // CH
    nsup = nch // SUP
    base = wid * nch
    _zero_rows(rows[0], CH, 8)
    _zero_shared(acc, rows[0], s)
    plsc.subcore_barrier()

    def issue_gather(b, slot, row):
      pltpu.async_copy(table.at[idxs_v.at[slot, row]], rows[b], gs[b])

    pltpu.sync_copy(src2.at[pl.ds(base, SUP)], idxs_v.at[0])
    pltpu.sync_copy(dst2.at[pl.ds(base, SUP)], idxd_v.at[0])
    for b in range(NB):
      issue_gather(b, 0, b)

    def sup_loop(t, carry):
      tp1 = t + 1
      slot = lax.rem(t, 2)
      nslot = lax.rem(tp1, 2)
      @pl.when(tp1 < nsup)
      def _():
        pltpu.async_copy(src2.at[pl.ds(base + tp1 * SUP, SUP)],
                         idxs_v.at[nslot], isem)
        pltpu.async_copy(dst2.at[pl.ds(base + tp1 * SUP, SUP)],
                         idxd_v.at[nslot], isem)
      for rloc in range(SUP // NB):
        for b in range(NB):
          row = rloc * NB + b
          pltpu.make_async_copy(table.at[pl.ds(0, CH)], rows[b],
                                gs[b]).wait()
          pltpu.async_copy(rows[b], acc.at[idxd_v.at[slot, row]], ss[b],
                           add=True)
        if rloc == SUP // NB - 1:
          @pl.when(tp1 < nsup)
          def _():
            pltpu.make_async_copy(src2.at[pl.ds(0, SUP)], idxs_v.at[0],
                                  isem).wait()
            pltpu.make_async_copy(dst2.at[pl.ds(0, SUP)], idxd_v.at[0],
                                  isem).wait()
        for b in range(NB):
          pltpu.make_async_copy(rows[b], acc.at[pl.ds(0, CH)], ss[b]).wait()
          if rloc < SUP // NB - 1:
            issue_gather(b, slot, rloc * NB + b + NB)
          else:
            @pl.when(tp1 < nsup)
            def _():
              issue_gather(b, nslot, b)
      return carry
    lax.fori_loop(0, nsup, sup_loop, 0)
    plsc.subcore_barrier()
    pltpu.sync_copy(acc.at[pl.ds(s * RPS, RPS)],
                    out.at[c, pl.ds(s * RPS, RPS)])
  return seg32


def _make_seg16(scaled):
  out_type = jax.ShapeDtypeStruct((NCORE, NP, 128), f32)
  nch = EPW16 // CH
  nsup = nch // SUP
  scratch = [
      pltpu.VMEM((2, SUP, CH), i32),
      pltpu.VMEM((2, SUP, CH), i32),
  ] + [pltpu.VMEM((CH, 128), f32)] * NB + [
      pltpu.VMEM_SHARED((NP, 128), f32),
  ] + [pltpu.SemaphoreType.DMA] * (2 * NB + 1)
  if scaled:
    out_type = (out_type, jax.ShapeDtypeStruct((NP,), f32))
    scratch += [pltpu.VMEM((CH,), f32)] * (3 * NB) + [
        pltpu.VMEM_SHARED((NP,), f32),
    ]

  def body(*args):
    if scaled:
      (t0, t1, es_h, ed_h, src2, dst2, out, den_out, *rest) = args
    else:
      t0, t1, src2, dst2, out, *rest = args
    idxs_v, idxd_v = rest[0], rest[1]
    rows = rest[2:2 + NB]
    acc = rest[2 + NB]
    gs = rest[3 + NB:3 + 2 * NB]
    ss = rest[3 + 2 * NB:3 + 3 * NB]
    isem = rest[3 + 3 * NB]
    if scaled:
      esb = rest[4 + 3 * NB:4 + 4 * NB]
      edb = rest[4 + 4 * NB:4 + 5 * NB]
      exb = rest[4 + 5 * NB:4 + 6 * NB]
      denacc = rest[4 + 6 * NB]
    c = lax.axis_index("c")
    s = lax.axis_index("s")
    base = s * nch
    _zero_rows(rows[0], CH, 8)
    _zero_shared(acc, rows[0], s)
    if scaled:
      for k in range(CH // 16):
        exb[0][pl.ds(k * 16, 16)] = jnp.zeros((16,), f32)
      for k in range(RPS // CH):
        pltpu.sync_copy(exb[0], denacc.at[pl.ds(s * RPS + k * CH, CH)])
    plsc.subcore_barrier()

    def issue_gather(b, slot, row):
      @pl.when(c == 0)
      def _():
        pltpu.async_copy(t0.at[idxs_v.at[slot, row]], rows[b], gs[b])
      @pl.when(c == 1)
      def _():
        pltpu.async_copy(t1.at[idxs_v.at[slot, row]], rows[b], gs[b])
      if scaled:
        pltpu.async_copy(es_h.at[idxs_v.at[slot, row]], esb[b], gs[b])
        pltpu.async_copy(ed_h.at[idxd_v.at[slot, row]], edb[b], gs[b])

    pltpu.sync_copy(src2.at[pl.ds(base, SUP)], idxs_v.at[0])
    pltpu.sync_copy(dst2.at[pl.ds(base, SUP)], idxd_v.at[0])
    for b in range(NB):
      issue_gather(b, 0, b)

    def sup_loop(t, carry):
      tp1 = t + 1
      slot = lax.rem(t, 2)
      nslot = lax.rem(tp1, 2)
      @pl.when(tp1 < nsup)
      def _():
        pltpu.async_copy(src2.at[pl.ds(base + tp1 * SUP, SUP)],
                         idxs_v.at[nslot], isem)
        pltpu.async_copy(dst2.at[pl.ds(base + tp1 * SUP, SUP)],
                         idxd_v.at[nslot], isem)
      for rloc in range(SUP // NB):
        for b in range(NB):
          row = rloc * NB + b
          pltpu.make_async_copy(t0.at[pl.ds(0, CH)], rows[b], gs[b]).wait()
          if scaled:
            pltpu.make_async_copy(es_h.at[pl.ds(0, CH)], esb[b],
                                  gs[b]).wait()
            pltpu.make_async_copy(es_h.at[pl.ds(0, CH)], edb[b],
                                  gs[b]).wait()
            def scale_grp(g, carry3, b=b):
              e = esb[b][pl.ds(g * 16, 16)] + edb[b][pl.ds(g * 16, 16)]
              e = jnp.where(e > 0, e, 0.2 * e)
              ex = jnp.exp(jnp.minimum(e, 50.0))
              exb[b][pl.ds(g * 16, 16)] = ex
              for jj in range(16):
                rr = g * 16 + jj
                bc = ex.at[jnp.full((16,), jj, i32)].get(
                    mode="promise_in_bounds")
                for k in range(8):
                  rows[b][rr, pl.ds(k * 16, 16)] = (
                      rows[b][rr, pl.ds(k * 16, 16)] * bc)
              return carry3
            lax.fori_loop(0, CH // 16, scale_grp, 0)
            @pl.when(c == 0)
            def _():
              pltpu.async_copy(exb[b], denacc.at[idxd_v.at[slot, row]],
                               ss[b], add=True)
          pltpu.async_copy(rows[b], acc.at[idxd_v.at[slot, row]], ss[b],
                           add=True)
        if rloc == SUP // NB - 1:
          @pl.when(tp1 < nsup)
          def _():
            pltpu.make_async_copy(src2.at[pl.ds(0, SUP)], idxs_v.at[0],
                                  isem).wait()
            pltpu.make_async_copy(dst2.at[pl.ds(0, SUP)], idxd_v.at[0],
                                  isem).wait()
        for b in range(NB):
          pltpu.make_async_copy(rows[b], acc.at[pl.ds(0, CH)], ss[b]).wait()
          if scaled:
            @pl.when(c == 0)
            def _():
              pltpu.make_async_copy(exb[b], denacc.at[pl.ds(0, CH)],
                                    ss[b]).wait()
          if rloc < SUP // NB - 1:
            issue_gather(b, slot, rloc * NB + b + NB)
          else:
            @pl.when(tp1 < nsup)
            def _():
              issue_gather(b, nslot, b)
      return carry
    lax.fori_loop(0, nsup, sup_loop, 0)
    plsc.subcore_barrier()
    pltpu.sync_copy(acc.at[pl.ds(s * RPS, RPS)],
                    out.at[c, pl.ds(s * RPS, RPS)])
    if scaled:
      @pl.when(c == 0)
      def _():
        pltpu.sync_copy(denacc.at[pl.ds(s * RPS, RPS)],
                        den_out.at[pl.ds(s * RPS, RPS)])

  return functools.partial(
      pl.kernel, out_type=out_type, mesh=_mesh(), scratch_types=scratch)(body)


def _rs2(cols):
  return pl.BlockSpec((R, cols), lambda i: (i, 0))


def _rs1():
  return pl.BlockSpec((R,), lambda i: (i,))


def _rs16():
  return pl.BlockSpec((R, 16), lambda i: (i, 0))


def _part(cidx):
  return pl.BlockSpec((1, R, 128), lambda i, c=cidx: (c, i, 0))


def _full2(a, b):
  return pl.BlockSpec((a, b), lambda i: (0, 0))


def _full1(n):
  return pl.BlockSpec((n,), lambda i: (0,))


def _dinv(da, db):
  deg = da + db
  return jnp.where(deg > 0, lax.rsqrt(deg), 0.0)


def _tc1_body(x_ref, w_ref, da_ref, db_ref, o_ref):
  dinv = _dinv(da_ref[...], db_ref[...])
  o_ref[...] = dinv[:, None] * jnp.dot(
      x_ref[...], w_ref[...], preferred_element_type=f32)


def _tc2_body(a0, a1, da, db, b_ref, w_ref, o_ref):
  dinv = _dinv(da[...], db[...])
  h = jnp.maximum(dinv[:, None] * (a0[0] + a1[0]) + b_ref[...][None, :], 0.0)
  o_ref[...] = dinv[:, None] * jnp.dot(
      h, w_ref[...], preferred_element_type=f32)


def _tc3_body(a0, a1, da, db, b_ref, w_ref, as_ref, ad_ref,
              t0_ref, t1_ref, es_ref, ed_ref):
  dinv = _dinv(da[...], db[...])
  h = jnp.maximum(dinv[:, None] * (a0[0] + a1[0]) + b_ref[...][None, :], 0.0)
  xw = jnp.dot(h, w_ref[...], preferred_element_type=f32)
  t0_ref[...] = xw[:, :128]
  t1_ref[...] = xw[:, 128:]
  es_ref[...] = jnp.sum(xw * as_ref[...][None, :], axis=1)
  ed_ref[...] = jnp.sum(xw * ad_ref[...][None, :], axis=1)


def _tc4_body(a0, a1, den_ref, bg_ref, w_ref, as_ref, ad_ref,
              t0_ref, t1_ref, es_ref, ed_ref):
  den = den_ref[...][:, None]
  h = jnp.concatenate([a0[0], a1[0]], axis=1) / (den + 1e-16)
  h = jnp.maximum(h + bg_ref[...][None, :], 0.0)
  xw = jnp.dot(h, w_ref[...], preferred_element_type=f32)
  t0_ref[...] = xw[:, :128]
  t1_ref[...] = xw[:, 128:]
  es_ref[...] = jnp.sum(xw * as_ref[...][None, :], axis=1)
  ed_ref[...] = jnp.sum(xw * ad_ref[...][None, :], axis=1)


def _tc5_body(a0, a1, den_ref, bg_ref, h0_ref, h1_ref):
  den = den_ref[...][:, None]
  h = jnp.concatenate([a0[0], a1[0]], axis=1) / (den + 1e-16)
  h = jnp.maximum(h + bg_ref[...][None, :], 0.0)
  h0_ref[...] = h[:, :128]
  h1_ref[...] = h[:, 128:]


def _tc6_body(h0, h1, a0, a1, w3a, b3a, w3b, b3b, o_ref):
  g = jnp.concatenate([h0[...] + a0[0], h1[...] + a1[0]], axis=1)
  g = jnp.maximum(jnp.dot(g, w3a[...], preferred_element_type=f32)
                  + b3a[...][None, :], 0.0)
  g = jnp.maximum(jnp.dot(g, w3b[...], preferred_element_type=f32)
                  + b3b[...][None, :], 0.0)
  o_ref[...] = g


def _tc7_body(g_ref, batch_ref, wout_ref, bout_ref, o_ref, sums, cnt):
  i = pl.program_id(0)
  @pl.when(i == 0)
  def _():
    sums[...] = jnp.zeros((B, 512), f32)
    cnt[...] = jnp.zeros((B, 128), f32)
  bi = batch_ref[...]
  iota = lax.broadcasted_iota(i32, (B, R), 0)
  mask = (bi[None, :] == iota).astype(f32)
  sums[...] += jnp.dot(mask, g_ref[...], preferred_element_type=f32)
  cnt[...] += jnp.broadcast_to(jnp.sum(mask, axis=1)[:, None], (B, 128))
  @pl.when(i == GRID - 1)
  def _():
    mean = sums[...] / jnp.maximum(cnt[...][:, 0:1], 1.0)
    o_ref[...] = jnp.dot(mean, wout_ref[...],
                         preferred_element_type=f32) + bout_ref[...][None, :]


def _sds(*shape):
  return jax.ShapeDtypeStruct(shape, f32)


def kernel(x, edge_index, batch, W1, b1, W2, b2, Wg1, bg1, as1, ad1,
           Wg2, bg2, as2, ad2, W3a, b3a, W3b, b3b, Wout, bout):
  loop = jnp.arange(N, dtype=i32)
  pad2 = E2P - E2
  src = jnp.concatenate(
      [edge_index[0], loop, jnp.zeros((pad2,), i32)]).reshape(E2P // CH, CH)
  dst = jnp.concatenate(
      [edge_index[1], loop, jnp.full((pad2,), N, i32)]).reshape(E2P // CH, CH)
  padg = E2P - E
  srcg = jnp.concatenate(
      [edge_index[0], jnp.zeros((padg,), i32)]).reshape(E2P // CH, CH)
  dstg = jnp.concatenate(
      [edge_index[1], jnp.full((padg,), N, i32)]).reshape(E2P // CH, CH)
  x_pad = jnp.pad(x, ((0, NP - N), (0, 0)))
  batch_pad = jnp.concatenate([batch, jnp.full((NP - N,), B, i32)])

  deg_fn = _make_deg()
  seg32 = _make_seg32()
  gat16 = _make_seg16(scaled=True)
  gin16 = _make_seg16(scaled=False)

  dega, degb = deg_fn(dst)

  xs1 = pl.pallas_call(
      _tc1_body, grid=(GRID,),
      in_specs=[_rs2(128), _full2(128, 128), _rs1(), _rs1()],
      out_specs=_rs2(128), out_shape=_sds(NP, 128),
  )(x_pad, W1, dega, degb)
  agg1 = seg32(xs1, src, dst)

  xs2 = pl.pallas_call(
      _tc2_body, grid=(GRID,),
      in_specs=[_part(0), _part(1), _rs1(), _rs1(), _full1(128),
                _full2(128, 128)],
      out_specs=_rs2(128), out_shape=_sds(NP, 128),
  )(agg1, agg1, dega, degb, b1, W2)
  agg2 = seg32(xs2, src, dst)

  t3a, t3b, es3, ed3 = pl.pallas_call(
      _tc3_body, grid=(GRID,),
      in_specs=[_part(0), _part(1), _rs1(), _rs1(), _full1(128),
                _full2(128, 256), _full1(256), _full1(256)],
      out_specs=[_rs2(128), _rs2(128), _rs1(), _rs1()],
      out_shape=[_sds(NP, 128), _sds(NP, 128), _sds(NP), _sds(NP)],
  )(agg2, agg2, dega, degb, b2, Wg1, as1, ad1)
  agg3, den3 = gat16(t3a, t3b, es3, ed3, src, dst)

  t4a, t4b, es4, ed4 = pl.pallas_call(
      _tc4_body, grid=(GRID,),
      in_specs=[_part(0), _part(1), _rs1(), _full1(256), _full2(256, 256),
                _full1(256), _full1(256)],
      out_specs=[_rs2(128), _rs2(128), _rs1(), _rs1()],
      out_shape=[_sds(NP, 128), _sds(NP, 128), _sds(NP), _sds(NP)],
  )(agg3, agg3, den3, bg1, Wg2, as2, ad2)
  agg4, den4 = gat16(t4a, t4b, es4, ed4, src, dst)

  h4a, h4b = pl.pallas_call(
      _tc5_body, grid=(GRID,),
      in_specs=[_part(0), _part(1), _rs1(), _full1(256)],
      out_specs=[_rs2(128), _rs2(128)],
      out_shape=[_sds(NP, 128), _sds(NP, 128)],
  )(agg4, agg4, den4, bg2)

  agg5 = gin16(h4a, h4b, srcg, dstg)

  g3 = pl.pallas_call(
      _tc6_body, grid=(GRID,),
      in_specs=[_rs2(128), _rs2(128), _part(0), _part(1),
                _full2(256, 512), _full1(512), _full2(512, 512), _full1(512)],
      out_specs=_rs2(512), out_shape=_sds(NP, 512),
  )(h4a, h4b, agg5, agg5, W3a, b3a, W3b, b3b)

  out = pl.pallas_call(
      _tc7_body, grid=(GRID,),
      in_specs=[_rs2(512), _rs1(), _full2(512, 16), _full1(16)],
      out_specs=pl.BlockSpec((B, 16), lambda i: (0, 0)),
      out_shape=_sds(B, 16),
      scratch_shapes=[pltpu.VMEM((B, 512), f32), pltpu.VMEM((B, 128), f32)],
  )(g3, batch_pad, Wout, bout)
  return out
